# baseline (speedup 1.0000x reference)
import jax
import jax.numpy as jnp
from jax.experimental import pallas as pl
from jax.experimental.pallas import tpu as pltpu


def _layernorm(x, g, b, eps=1e-5):
    m = jnp.mean(x, axis=-1, keepdims=True)
    v = jnp.var(x, axis=-1, keepdims=True)
    return (x - m) / jnp.sqrt(v + eps) * g + b


def _gat_conv(x, src, dst, W, a_src, a_dst, bias, heads, dph):
    n = x.shape[0]
    loop = jnp.arange(n, dtype=src.dtype)
    s = jnp.concatenate([src, loop])
    d = jnp.concatenate([dst, loop])
    h = (x @ W).reshape(n, heads, dph)
    al_s = jnp.sum(h * a_src[None, :, :], axis=-1)
    al_d = jnp.sum(h * a_dst[None, :, :], axis=-1)
    alpha = al_s[s] + al_d[d]
    alpha = jax.nn.leaky_relu(alpha, negative_slope=0.2)
    amax = jax.ops.segment_max(alpha, d, num_segments=n)
    amax = jnp.where(jnp.isfinite(amax), amax, 0.0)
    ex = jnp.exp(alpha - amax[d])
    den = jax.ops.segment_sum(ex, d, num_segments=n)
    coef = ex / (den[d] + 1e-16)
    msg = h[s] * coef[:, :, None]
    out = jax.ops.segment_sum(msg, d, num_segments=n)
    return out.reshape(n, heads * dph) + bias


def _gat_block(x, src, dst, p, heads, dph):
    h = _gat_conv(x, src, dst, p['W'], p['a_src'], p['a_dst'], p['b'], heads, dph)
    h = _layernorm(h, p['ln_g'], p['ln_b'])
    return jax.nn.gelu(h)


def _head_kernel(gf_ref, u_ref, *refs):
    p_refs = refs[:-1]
    o_ref = refs[-1]
    (ge_W1, ge_b1, ge_g1, ge_be1, ge_W2, ge_b2, ge_g2, ge_be2,
     fu_W1, fu_b1, fu_g1, fu_be1, fu_W2, fu_b2, fu_g2, fu_be2,
     cl_W1, cl_b1, cl_W2, cl_b2) = p_refs
    u = u_ref[...]
    g = u @ ge_W1[...] + ge_b1[...][None, :]
    g = jax.nn.gelu(_layernorm(g, ge_g1[...], ge_be1[...]))
    g = g @ ge_W2[...] + ge_b2[...][None, :]
    g = jax.nn.gelu(_layernorm(g, ge_g2[...], ge_be2[...]))
    combined = jnp.concatenate([gf_ref[...], g], axis=1)
    f = combined @ fu_W1[...] + fu_b1[...][None, :]
    f = jax.nn.gelu(_layernorm(f, fu_g1[...], fu_be1[...]))
    f = f @ fu_W2[...] + fu_b2[...][None, :]
    f = jax.nn.gelu(_layernorm(f, fu_g2[...], fu_be2[...]))
    c = jax.nn.gelu(f @ cl_W1[...] + cl_b1[...][None, :])
    o_ref[...] = c @ cl_W2[...] + cl_b2[...][None, :]


def kernel(x, edge_index, batch, u, params):
    src = edge_index[0]
    dst = edge_index[1]
    B = u.shape[0]
    h = x @ params['ne_W'] + params['ne_b']
    h = _gat_block(h, src, dst, params['g1'], 4, 32)
    h = _gat_block(h, src, dst, params['g2'], 4, 32)
    h = _gat_block(h, src, dst, params['g3'], 4, 16)
    ones = jnp.ones((h.shape[0],), jnp.float32)
    cnt = jax.ops.segment_sum(ones, batch, num_segments=B)
    graph_feat = jax.ops.segment_sum(h, batch, num_segments=B) / jnp.maximum(cnt, 1.0)[:, None]

    pk = ['ge_W1', 'ge_b1', 'ge_g1', 'ge_be1', 'ge_W2', 'ge_b2', 'ge_g2', 'ge_be2',
          'fu_W1', 'fu_b1', 'fu_g1', 'fu_be1', 'fu_W2', 'fu_b2', 'fu_g2', 'fu_be2',
          'cl_W1', 'cl_b1', 'cl_W2', 'cl_b2']
    plist = [params[k] for k in pk]
    logits = pl.pallas_call(
        _head_kernel,
        out_shape=jax.ShapeDtypeStruct((B, 2), jnp.float32),
    )(graph_feat, u, *plist)
    return logits


# speedup vs baseline: 22.5215x; 22.5215x over previous
import functools

import jax
import jax.numpy as jnp
from jax import lax
from jax.experimental import pallas as pl
from jax.experimental.pallas import tpu as pltpu
from jax.experimental.pallas import tpu_sc as plsc

NC = 2
NS = 16
NW = NC * NS
L = 16
K = 80


def _mesh():
    return plsc.VectorSubcoreMesh(core_axis_name="c", subcore_axis_name="s")


def _sc_pass_a(s_arr, d_arr, als, ald, m, n, e):
    ew = e // NW
    nchunk = ew // K
    zrows = 400
    nblk = n // zrows

    @functools.partial(
        pl.kernel,
        out_type=(jax.ShapeDtypeStruct((e, 16), jnp.float32),
                  jax.ShapeDtypeStruct((NC, n, 16), jnp.float32)),
        mesh=_mesh(),
        compiler_params=pltpu.CompilerParams(use_tc_tiling_on_sc=False),
        scratch_types=[
            pltpu.VMEM((1, K), jnp.int32),
            pltpu.VMEM((1, K), jnp.int32),
            pltpu.VMEM((K, 16), jnp.float32),
            pltpu.VMEM((K, 16), jnp.float32),
            pltpu.VMEM((K, 16), jnp.float32),
            pltpu.VMEM((zrows, 16), jnp.float32),
            pltpu.VMEM((16,), jnp.float32),
            pltpu.VMEM_SHARED((n, 16), jnp.float32),
            pltpu.SemaphoreType.DMA,
            pltpu.SemaphoreType.DMA,
        ],
    )
    def run(s_hbm, d_hbm, als_hbm, ald_hbm, m_hbm, ex_hbm, den_hbm,
            sv, dv, rs, rd, denrows, zbuf, mv, densh, sem1, sem2):
        c = lax.axis_index("c")
        s = lax.axis_index("s")
        wid = s * NC + c
        pltpu.sync_copy(m_hbm, mv)
        zv = jnp.zeros((L,), jnp.float32)

        def zloop(i, _):
            zbuf[i, :] = zv
            return 0
        lax.fori_loop(0, zrows, zloop, 0)
        for r in range(K):
            denrows[r, :] = zv
        for bb in range((nblk + NS - 1) // NS):
            blk = bb * NS + s
            @pl.when(blk < nblk)
            def _():
                pltpu.sync_copy(zbuf, densh.at[pl.ds(blk * zrows, zrows)])
        plsc.subcore_barrier()

        iot = lax.iota(jnp.int32, L)
        lanemask = iot < 4
        mvec = mv[...]

        def chunk(i, _):
            base = wid * ew + i * K
            pltpu.sync_copy(s_hbm.at[pl.ds(base, K)], sv.at[0])
            pltpu.sync_copy(d_hbm.at[pl.ds(base, K)], dv.at[0])
            cp1 = pltpu.async_copy(als_hbm.at[sv.at[0]], rs, sem1)
            cp2 = pltpu.async_copy(ald_hbm.at[dv.at[0]], rd, sem2)
            cp1.wait()
            cp2.wait()
            for j in range(K):
                a = rs[j, :] + rd[j, :]
                a = jnp.where(a >= 0.0, a, a * jnp.float32(0.2))
                exf = jnp.exp(a - mvec)
                denrows[j, :] = jnp.where(lanemask, exf, jnp.float32(0.0))
            pltpu.sync_copy(denrows, ex_hbm.at[pl.ds(base, K)])
            pltpu.sync_copy(denrows, densh.at[dv.at[0]], add=True)
            return 0
        lax.fori_loop(0, nchunk, chunk, 0)

        plsc.subcore_barrier()
        for bb in range((nblk + NS - 1) // NS):
            blk = bb * NS + s
            @pl.when(blk < nblk)
            def _():
                pltpu.sync_copy(densh.at[pl.ds(blk * zrows, zrows)], zbuf)
                pltpu.sync_copy(zbuf, den_hbm.at[c, pl.ds(blk * zrows, zrows)])

    return run(s_arr, d_arr, als, ald, m)


def _sc_pass_b(s_arr, d_arr, ex, hw, n, e, dph):
    ew = e // NW
    nchunk = ew // K
    zrows = 400
    nblk = n // zrows

    @functools.partial(
        pl.kernel,
        out_type=jax.ShapeDtypeStruct((NC, 4, n, dph), jnp.float32),
        mesh=_mesh(),
        compiler_params=pltpu.CompilerParams(use_tc_tiling_on_sc=False),
        scratch_types=[
            pltpu.VMEM((1, K), jnp.int32),
            pltpu.VMEM((1, K), jnp.int32),
            pltpu.VMEM((1, K), jnp.int32),
            pltpu.VMEM((1, K), jnp.float32),
            pltpu.VMEM((K, dph), jnp.float32),
            pltpu.VMEM((zrows, dph), jnp.float32),
            pltpu.VMEM_SHARED((n, dph), jnp.float32),
            pltpu.SemaphoreType.DMA,
        ],
    )
    def run(s_hbm, d_hbm, ex_hbm, hw_hbm, out_hbm,
            sv, dv, sidx, exv, rows, zbuf, acc, sem1):
        c = lax.axis_index("c")
        s = lax.axis_index("s")
        wid = s * NC + c
        zv = jnp.zeros((L,), jnp.float32)
        iot = lax.iota(jnp.int32, L)

        def zloop(i, _):
            for q in range(dph // L):
                zbuf[i, pl.ds(q * L, L)] = zv
            return 0

        for h in range(4):
            lax.fori_loop(0, zrows, zloop, 0)
            for bb in range((nblk + NS - 1) // NS):
                blk = bb * NS + s
                @pl.when(blk < nblk)
                def _():
                    pltpu.sync_copy(zbuf, acc.at[pl.ds(blk * zrows, zrows)])
            plsc.subcore_barrier()

            def chunk(i, _):
                base = wid * ew + i * K
                pltpu.sync_copy(s_hbm.at[pl.ds(base, K)], sv.at[0])
                pltpu.sync_copy(d_hbm.at[pl.ds(base, K)], dv.at[0])
                for g in range(K // L):
                    sidx[0, pl.ds(g * L, L)] = sv[0, pl.ds(g * L, L)] + jnp.int32(h * n)
                pltpu.async_copy(hw_hbm.at[sidx.at[0]], rows, sem1).wait()
                pltpu.sync_copy(ex_hbm.at[pl.ds(h * e + base, K)], exv.at[0])

                for g in range(K // L):
                    exg = exv[0, pl.ds(g * L, L)]
                    for j2 in range(L):
                        j = g * L + j2
                        exj = exg[j2]
                        for q in range(dph // L):
                            rows[j, pl.ds(q * L, L)] = rows[j, pl.ds(q * L, L)] * exj
                pltpu.sync_copy(rows, acc.at[dv.at[0]], add=True)
                return 0
            lax.fori_loop(0, nchunk, chunk, 0)

            plsc.subcore_barrier()
            for bb in range((nblk + NS - 1) // NS):
                blk = bb * NS + s
                @pl.when(blk < nblk)
                def _():
                    pltpu.sync_copy(acc.at[pl.ds(blk * zrows, zrows)], zbuf)
                    pltpu.sync_copy(zbuf, out_hbm.at[c, h, pl.ds(blk * zrows, zrows)])
            plsc.subcore_barrier()

    return run(s_arr, d_arr, ex, hw)


def _layernorm(x, g, b, eps=1e-5):
    m = jnp.mean(x, axis=-1, keepdims=True)
    v = jnp.var(x, axis=-1, keepdims=True)
    return (x - m) / jnp.sqrt(v + eps) * g + b


def _gat_block(h_in, s_arr, d_arr, p, heads, dph):
    n = h_in.shape[0]
    e = s_arr.shape[0]
    hw = h_in @ p['W']
    h4 = hw.reshape(n, heads, dph)
    al_s = jnp.sum(h4 * p['a_src'][None, :, :], axis=-1)
    al_d = jnp.sum(h4 * p['a_dst'][None, :, :], axis=-1)
    ms = jnp.max(al_s, axis=0) + jnp.max(al_d, axis=0)
    m = jnp.where(ms >= 0.0, ms, 0.2 * ms)
    pad12 = jnp.zeros((n, 12), jnp.float32)
    als16 = jnp.concatenate([al_s, pad12], axis=1)
    ald16 = jnp.concatenate([al_d, pad12], axis=1)
    m16 = jnp.concatenate([m, jnp.zeros((12,), jnp.float32)])

    exr, den_p = _sc_pass_a(s_arr, d_arr, als16, ald16, m16, n, e)
    ext = jnp.transpose(exr[:, :4], (1, 0)).reshape(4 * e)
    hw_hm = jnp.transpose(h4, (1, 0, 2)).reshape(heads * n, dph)
    out_p = _sc_pass_b(s_arr, d_arr, ext, hw_hm, n, e, dph)

    alpha_self = al_s + al_d
    alpha_self = jnp.where(alpha_self >= 0.0, alpha_self, 0.2 * alpha_self)
    ex_self = jnp.exp(alpha_self - m[None, :])
    den = den_p[0, :, :4] + den_p[1, :, :4] + ex_self
    num = out_p[0] + out_p[1]
    num = jnp.transpose(num, (1, 0, 2)) + ex_self[:, :, None] * h4
    out = num / (den[:, :, None] + 1e-16)
    out = out.reshape(n, heads * dph) + p['b']
    out = _layernorm(out, p['ln_g'], p['ln_b'])
    return jax.nn.gelu(out)


def _head_kernel(gf_ref, u_ref, *refs):
    p_refs = refs[:-1]
    o_ref = refs[-1]
    (ge_W1, ge_b1, ge_g1, ge_be1, ge_W2, ge_b2, ge_g2, ge_be2,
     fu_W1, fu_b1, fu_g1, fu_be1, fu_W2, fu_b2, fu_g2, fu_be2,
     cl_W1, cl_b1, cl_W2, cl_b2) = p_refs
    u = u_ref[...]
    g = u @ ge_W1[...] + ge_b1[...][None, :]
    g = jax.nn.gelu(_layernorm(g, ge_g1[...], ge_be1[...]))
    g = g @ ge_W2[...] + ge_b2[...][None, :]
    g = jax.nn.gelu(_layernorm(g, ge_g2[...], ge_be2[...]))
    combined = jnp.concatenate([gf_ref[...], g], axis=1)
    f = combined @ fu_W1[...] + fu_b1[...][None, :]
    f = jax.nn.gelu(_layernorm(f, fu_g1[...], fu_be1[...]))
    f = f @ fu_W2[...] + fu_b2[...][None, :]
    f = jax.nn.gelu(_layernorm(f, fu_g2[...], fu_be2[...]))
    c = jax.nn.gelu(f @ cl_W1[...] + cl_b1[...][None, :])
    o_ref[...] = c @ cl_W2[...] + cl_b2[...][None, :]


def kernel(x, edge_index, batch, u, params):
    s_arr = edge_index[0]
    d_arr = edge_index[1]
    B = u.shape[0]
    h = x @ params['ne_W'] + params['ne_b']
    h = _gat_block(h, s_arr, d_arr, params['g1'], 4, 32)
    h = _gat_block(h, s_arr, d_arr, params['g2'], 4, 32)
    h = _gat_block(h, s_arr, d_arr, params['g3'], 4, 16)
    ones = jnp.ones((h.shape[0],), jnp.float32)
    cnt = jax.ops.segment_sum(ones, batch, num_segments=B)
    graph_feat = jax.ops.segment_sum(h, batch, num_segments=B) / jnp.maximum(cnt, 1.0)[:, None]

    pk = ['ge_W1', 'ge_b1', 'ge_g1', 'ge_be1', 'ge_W2', 'ge_b2', 'ge_g2', 'ge_be2',
          'fu_W1', 'fu_b1', 'fu_g1', 'fu_be1', 'fu_W2', 'fu_b2', 'fu_g2', 'fu_be2',
          'cl_W1', 'cl_b1', 'cl_W2', 'cl_b2']
    plist = [params[k] for k in pk]
    logits = pl.pallas_call(
        _head_kernel,
        out_shape=jax.ShapeDtypeStruct((B, 2), jnp.float32),
    )(graph_feat, u, *plist)
    return logits


# speedup vs baseline: 38.5990x; 1.7139x over previous
import functools

import jax
import jax.numpy as jnp
from jax import lax
from jax.experimental import pallas as pl
from jax.experimental.pallas import tpu as pltpu
from jax.experimental.pallas import tpu_sc as plsc

NC = 2
NS = 16
NW = NC * NS
L = 16
K = 80


def _mesh():
    return plsc.VectorSubcoreMesh(core_axis_name="c", subcore_axis_name="s")


def _sc_pass_a(s_arr, d_arr, als, ald, m, n, e):
    ew = e // NW
    nchunk = ew // K
    zrows = 400
    nblk = n // zrows

    @functools.partial(
        pl.kernel,
        out_type=(jax.ShapeDtypeStruct((e, 16), jnp.float32),
                  jax.ShapeDtypeStruct((NC, n, 16), jnp.float32)),
        mesh=_mesh(),
        compiler_params=pltpu.CompilerParams(use_tc_tiling_on_sc=False),
        scratch_types=[
            pltpu.VMEM((1, K), jnp.int32),
            pltpu.VMEM((1, K), jnp.int32),
            pltpu.VMEM((K, 16), jnp.float32),
            pltpu.VMEM((K, 16), jnp.float32),
            pltpu.VMEM((K, 16), jnp.float32),
            pltpu.VMEM((zrows, 16), jnp.float32),
            pltpu.VMEM((16,), jnp.float32),
            pltpu.VMEM_SHARED((n, 16), jnp.float32),
            pltpu.SemaphoreType.DMA,
            pltpu.SemaphoreType.DMA,
        ],
    )
    def run(s_hbm, d_hbm, als_hbm, ald_hbm, m_hbm, ex_hbm, den_hbm,
            sv, dv, rs, rd, denrows, zbuf, mv, densh, sem1, sem2):
        c = lax.axis_index("c")
        s = lax.axis_index("s")
        wid = s * NC + c
        pltpu.sync_copy(m_hbm, mv)
        zv = jnp.zeros((L,), jnp.float32)

        def zloop(i, _):
            zbuf[i, :] = zv
            return 0
        lax.fori_loop(0, zrows, zloop, 0)
        for r in range(K):
            denrows[r, :] = zv
        for bb in range((nblk + NS - 1) // NS):
            blk = bb * NS + s
            @pl.when(blk < nblk)
            def _():
                pltpu.sync_copy(zbuf, densh.at[pl.ds(blk * zrows, zrows)])
        plsc.subcore_barrier()

        iot = lax.iota(jnp.int32, L)
        lanemask = iot < 4
        mvec = mv[...]

        def chunk(i, _):
            base = wid * ew + i * K
            pltpu.sync_copy(s_hbm.at[pl.ds(base, K)], sv.at[0])
            pltpu.sync_copy(d_hbm.at[pl.ds(base, K)], dv.at[0])
            cp1 = pltpu.async_copy(als_hbm.at[sv.at[0]], rs, sem1)
            cp2 = pltpu.async_copy(ald_hbm.at[dv.at[0]], rd, sem2)
            cp1.wait()
            cp2.wait()
            for j in range(K):
                a = rs[j, :] + rd[j, :]
                a = jnp.where(a >= 0.0, a, a * jnp.float32(0.2))
                exf = jnp.exp(a - mvec)
                denrows[j, :] = jnp.where(lanemask, exf, jnp.float32(0.0))
            pltpu.sync_copy(denrows, ex_hbm.at[pl.ds(base, K)])
            pltpu.sync_copy(denrows, densh.at[dv.at[0]], add=True)
            return 0
        lax.fori_loop(0, nchunk, chunk, 0)

        plsc.subcore_barrier()
        for bb in range((nblk + NS - 1) // NS):
            blk = bb * NS + s
            @pl.when(blk < nblk)
            def _():
                pltpu.sync_copy(densh.at[pl.ds(blk * zrows, zrows)], zbuf)
                pltpu.sync_copy(zbuf, den_hbm.at[c, pl.ds(blk * zrows, zrows)])

    return run(s_arr, d_arr, als, ald, m)


def _sc_pass_b(s_arr, d_arr, ex, hw, n, e, dph):
    ew = e // NW
    nchunk = ew // K
    zrows = 400
    nblk = n // zrows

    @functools.partial(
        pl.kernel,
        out_type=jax.ShapeDtypeStruct((NC, 4, n, dph), jnp.float32),
        mesh=_mesh(),
        compiler_params=pltpu.CompilerParams(use_tc_tiling_on_sc=False),
        scratch_types=[
            pltpu.VMEM((2, 1, K), jnp.int32),
            pltpu.VMEM((2, 1, K), jnp.int32),
            pltpu.VMEM((2, 1, K), jnp.int32),
            pltpu.VMEM((2, 1, K), jnp.int32),
            pltpu.VMEM((2, 1, K), jnp.float32),
            pltpu.VMEM((2, K, dph), jnp.float32),
            pltpu.VMEM((zrows, dph), jnp.float32),
            pltpu.VMEM_SHARED((n, dph), jnp.float32),
            pltpu.SemaphoreType.DMA,
            pltpu.SemaphoreType.DMA,
            pltpu.SemaphoreType.DMA,
            pltpu.SemaphoreType.DMA,
            pltpu.SemaphoreType.DMA,
            pltpu.SemaphoreType.DMA,
            pltpu.SemaphoreType.DMA,
            pltpu.SemaphoreType.DMA,
        ],
    )
    def run(s_hbm, d_hbm, ex_hbm, hw_hbm, out_hbm,
            sv, dv, sidx, sdx, exv, rows, zbuf, acc,
            semi0, semi1, semg0, semg1, seme0, seme1, sems0, sems1):
        c = lax.axis_index("c")
        s = lax.axis_index("s")
        wid = s * NC + c
        zv = jnp.zeros((L,), jnp.float32)
        semi = (semi0, semi1)
        semg = (semg0, semg1)
        seme = (seme0, seme1)
        sems = (sems0, sems1)

        def zloop(i, _):
            for q in range(dph // L):
                zbuf[i, pl.ds(q * L, L)] = zv
            return 0

        def issue_idx(i, p):
            base = wid * ew + i * K
            pltpu.async_copy(s_hbm.at[pl.ds(base, K)], sv.at[p, 0], semi[p])
            pltpu.async_copy(d_hbm.at[pl.ds(base, K)], dv.at[p, 0], semi[p])

        def wait_idx(i, p):
            base = wid * ew + i * K
            pltpu.make_async_copy(s_hbm.at[pl.ds(base, K)], sv.at[p, 0], semi[p]).wait()
            pltpu.make_async_copy(d_hbm.at[pl.ds(base, K)], dv.at[p, 0], semi[p]).wait()

        def issue_gather_ex(i, p, h):
            base = wid * ew + i * K
            for g in range(K // L):
                sidx[p, 0, pl.ds(g * L, L)] = sv[p, 0, pl.ds(g * L, L)] + jnp.int32(h * n)
            pltpu.async_copy(hw_hbm.at[sidx.at[p, 0]], rows.at[p], semg[p])
            pltpu.async_copy(ex_hbm.at[pl.ds(h * e + i * K + wid * ew, K)],
                             exv.at[p, 0], seme[p])

        def wait_gather_ex(i, p, h):
            base = wid * ew + i * K
            pltpu.make_async_copy(hw_hbm.at[sidx.at[p, 0]], rows.at[p], semg[p]).wait()
            pltpu.make_async_copy(ex_hbm.at[pl.ds(h * e + base, K)],
                                  exv.at[p, 0], seme[p]).wait()

        def wait_scatter(p):
            pltpu.make_async_copy(rows.at[p], acc.at[sdx.at[p, 0]], sems[p]).wait()

        def compute_and_scatter(i, p):
            for g in range(K // L):
                exg = exv[p, 0, pl.ds(g * L, L)]
                for j2 in range(L):
                    j = g * L + j2
                    exj = exg[j2]
                    for q in range(dph // L):
                        rows[p, j, pl.ds(q * L, L)] = rows[p, j, pl.ds(q * L, L)] * exj
            for g in range(K // L):
                sdx[p, 0, pl.ds(g * L, L)] = dv[p, 0, pl.ds(g * L, L)]
            pltpu.async_copy(rows.at[p], acc.at[sdx.at[p, 0]], sems[p], add=True)

        for h in range(4):
            lax.fori_loop(0, zrows, zloop, 0)
            for bb in range((nblk + NS - 1) // NS):
                blk = bb * NS + s
                @pl.when(blk < nblk)
                def _():
                    pltpu.sync_copy(zbuf, acc.at[pl.ds(blk * zrows, zrows)])
            plsc.subcore_barrier()

            issue_idx(0, 0)
            wait_idx(0, 0)
            issue_gather_ex(0, 0, h)
            issue_idx(1, 1)

            def step(i, p, q, wait_sc):
                wait_idx(i + 1, q)
                if wait_sc:
                    wait_scatter(q)
                issue_gather_ex(i + 1, q, h)
                wait_gather_ex(i, p, h)
                compute_and_scatter(i, p)
                @pl.when(i + 2 < nchunk)
                def _():
                    issue_idx(i + 2, p)

            step(0, 0, 1, False)
            step(1, 1, 0, True)

            def pair_rest(t, _):
                i = 2 * t
                step(i, 0, 1, True)
                step(i + 1, 1, 0, True)
                return 0
            lax.fori_loop(1, (nchunk - 1) // 2, pair_rest, 0)
            wait_gather_ex(nchunk - 1, 0, h)
            compute_and_scatter(nchunk - 1, 0)
            wait_scatter(0)
            wait_scatter(1)

            plsc.subcore_barrier()
            for bb in range((nblk + NS - 1) // NS):
                blk = bb * NS + s
                @pl.when(blk < nblk)
                def _():
                    pltpu.sync_copy(acc.at[pl.ds(blk * zrows, zrows)], zbuf)
                    pltpu.sync_copy(zbuf, out_hbm.at[c, h, pl.ds(blk * zrows, zrows)])
            plsc.subcore_barrier()

    return run(s_arr, d_arr, ex, hw)


def _layernorm(x, g, b, eps=1e-5):
    m = jnp.mean(x, axis=-1, keepdims=True)
    v = jnp.var(x, axis=-1, keepdims=True)
    return (x - m) / jnp.sqrt(v + eps) * g + b


def _gat_block(h_in, s_arr, d_arr, p, heads, dph):
    n = h_in.shape[0]
    e = s_arr.shape[0]
    hw = h_in @ p['W']
    h4 = hw.reshape(n, heads, dph)
    al_s = jnp.sum(h4 * p['a_src'][None, :, :], axis=-1)
    al_d = jnp.sum(h4 * p['a_dst'][None, :, :], axis=-1)
    ms = jnp.max(al_s, axis=0) + jnp.max(al_d, axis=0)
    m = jnp.where(ms >= 0.0, ms, 0.2 * ms)
    pad12 = jnp.zeros((n, 12), jnp.float32)
    als16 = jnp.concatenate([al_s, pad12], axis=1)
    ald16 = jnp.concatenate([al_d, pad12], axis=1)
    m16 = jnp.concatenate([m, jnp.zeros((12,), jnp.float32)])

    exr, den_p = _sc_pass_a(s_arr, d_arr, als16, ald16, m16, n, e)
    ext = jnp.transpose(exr[:, :4], (1, 0)).reshape(4 * e)
    hw_hm = jnp.transpose(h4, (1, 0, 2)).reshape(heads * n, dph)
    out_p = _sc_pass_b(s_arr, d_arr, ext, hw_hm, n, e, dph)

    alpha_self = al_s + al_d
    alpha_self = jnp.where(alpha_self >= 0.0, alpha_self, 0.2 * alpha_self)
    ex_self = jnp.exp(alpha_self - m[None, :])
    den = den_p[0, :, :4] + den_p[1, :, :4] + ex_self
    num = out_p[0] + out_p[1]
    num = jnp.transpose(num, (1, 0, 2)) + ex_self[:, :, None] * h4
    out = num / (den[:, :, None] + 1e-16)
    out = out.reshape(n, heads * dph) + p['b']
    out = _layernorm(out, p['ln_g'], p['ln_b'])
    return jax.nn.gelu(out)


def _head_kernel(gf_ref, u_ref, *refs):
    p_refs = refs[:-1]
    o_ref = refs[-1]
    (ge_W1, ge_b1, ge_g1, ge_be1, ge_W2, ge_b2, ge_g2, ge_be2,
     fu_W1, fu_b1, fu_g1, fu_be1, fu_W2, fu_b2, fu_g2, fu_be2,
     cl_W1, cl_b1, cl_W2, cl_b2) = p_refs
    u = u_ref[...]
    g = u @ ge_W1[...] + ge_b1[...][None, :]
    g = jax.nn.gelu(_layernorm(g, ge_g1[...], ge_be1[...]))
    g = g @ ge_W2[...] + ge_b2[...][None, :]
    g = jax.nn.gelu(_layernorm(g, ge_g2[...], ge_be2[...]))
    combined = jnp.concatenate([gf_ref[...], g], axis=1)
    f = combined @ fu_W1[...] + fu_b1[...][None, :]
    f = jax.nn.gelu(_layernorm(f, fu_g1[...], fu_be1[...]))
    f = f @ fu_W2[...] + fu_b2[...][None, :]
    f = jax.nn.gelu(_layernorm(f, fu_g2[...], fu_be2[...]))
    c = jax.nn.gelu(f @ cl_W1[...] + cl_b1[...][None, :])
    o_ref[...] = c @ cl_W2[...] + cl_b2[...][None, :]


def kernel(x, edge_index, batch, u, params):
    s_arr = edge_index[0]
    d_arr = edge_index[1]
    B = u.shape[0]
    h = x @ params['ne_W'] + params['ne_b']
    h = _gat_block(h, s_arr, d_arr, params['g1'], 4, 32)
    h = _gat_block(h, s_arr, d_arr, params['g2'], 4, 32)
    h = _gat_block(h, s_arr, d_arr, params['g3'], 4, 16)
    ones = jnp.ones((h.shape[0],), jnp.float32)
    cnt = jax.ops.segment_sum(ones, batch, num_segments=B)
    graph_feat = jax.ops.segment_sum(h, batch, num_segments=B) / jnp.maximum(cnt, 1.0)[:, None]

    pk = ['ge_W1', 'ge_b1', 'ge_g1', 'ge_be1', 'ge_W2', 'ge_b2', 'ge_g2', 'ge_be2',
          'fu_W1', 'fu_b1', 'fu_g1', 'fu_be1', 'fu_W2', 'fu_b2', 'fu_g2', 'fu_be2',
          'cl_W1', 'cl_b1', 'cl_W2', 'cl_b2']
    plist = [params[k] for k in pk]
    logits = pl.pallas_call(
        _head_kernel,
        out_shape=jax.ShapeDtypeStruct((B, 2), jnp.float32),
    )(graph_feat, u, *plist)
    return logits


# speedup vs baseline: 44.6450x; 1.1566x over previous
import functools

import jax
import jax.numpy as jnp
from jax import lax
from jax.experimental import pallas as pl
from jax.experimental.pallas import tpu as pltpu
from jax.experimental.pallas import tpu_sc as plsc

NC = 2
NS = 16
NW = NC * NS
L = 16
K = 80


def _mesh():
    return plsc.VectorSubcoreMesh(core_axis_name="c", subcore_axis_name="s")


def _sc_pass_a(s_arr, d_arr, als, ald, m, n, e):
    ew = e // NW
    nchunk = ew // K
    zrows = 400
    nblk = n // zrows

    @functools.partial(
        pl.kernel,
        out_type=(jax.ShapeDtypeStruct((e, 16), jnp.float32),
                  jax.ShapeDtypeStruct((NC, n, 16), jnp.float32)),
        mesh=_mesh(),
        compiler_params=pltpu.CompilerParams(use_tc_tiling_on_sc=False),
        scratch_types=[
            pltpu.VMEM((2, 1, K), jnp.int32),
            pltpu.VMEM((2, 1, K), jnp.int32),
            pltpu.VMEM((2, 1, K), jnp.int32),
            pltpu.VMEM((2, K, 16), jnp.float32),
            pltpu.VMEM((2, K, 16), jnp.float32),
            pltpu.VMEM((2, K, 16), jnp.float32),
            pltpu.VMEM((zrows, 16), jnp.float32),
            pltpu.VMEM((16,), jnp.float32),
            pltpu.VMEM_SHARED((n, 16), jnp.float32),
            pltpu.SemaphoreType.DMA,
            pltpu.SemaphoreType.DMA,
            pltpu.SemaphoreType.DMA,
            pltpu.SemaphoreType.DMA,
            pltpu.SemaphoreType.DMA,
            pltpu.SemaphoreType.DMA,
            pltpu.SemaphoreType.DMA,
            pltpu.SemaphoreType.DMA,
        ],
    )
    def run(s_hbm, d_hbm, als_hbm, ald_hbm, m_hbm, ex_hbm, den_hbm,
            sv, dv, sdx, rs, rd, denrows, zbuf, mv, densh,
            semi0, semi1, semg0, semg1, semx0, semx1, sems0, sems1):
        c = lax.axis_index("c")
        s = lax.axis_index("s")
        wid = s * NC + c
        semi = (semi0, semi1)
        semg = (semg0, semg1)
        semx = (semx0, semx1)
        sems = (sems0, sems1)
        pltpu.sync_copy(m_hbm, mv)
        zv = jnp.zeros((L,), jnp.float32)

        def zloop(i, _):
            zbuf[i, :] = zv
            return 0
        lax.fori_loop(0, zrows, zloop, 0)
        for bb in range((nblk + NS - 1) // NS):
            blk = bb * NS + s
            @pl.when(blk < nblk)
            def _():
                pltpu.sync_copy(zbuf, densh.at[pl.ds(blk * zrows, zrows)])
        plsc.subcore_barrier()

        iot = lax.iota(jnp.int32, L)
        lanemask = iot < 4
        mvec = mv[...]

        def issue_idx(i, p):
            base = wid * ew + i * K
            pltpu.async_copy(s_hbm.at[pl.ds(base, K)], sv.at[p, 0], semi[p])
            pltpu.async_copy(d_hbm.at[pl.ds(base, K)], dv.at[p, 0], semi[p])

        def wait_idx(i, p):
            base = wid * ew + i * K
            pltpu.make_async_copy(s_hbm.at[pl.ds(base, K)], sv.at[p, 0], semi[p]).wait()
            pltpu.make_async_copy(d_hbm.at[pl.ds(base, K)], dv.at[p, 0], semi[p]).wait()

        def issue_gathers(i, p):
            pltpu.async_copy(als_hbm.at[sv.at[p, 0]], rs.at[p], semg[p])
            pltpu.async_copy(ald_hbm.at[dv.at[p, 0]], rd.at[p], semg[p])

        def wait_gathers(i, p):
            pltpu.make_async_copy(als_hbm.at[sv.at[p, 0]], rs.at[p], semg[p]).wait()
            pltpu.make_async_copy(ald_hbm.at[dv.at[p, 0]], rd.at[p], semg[p]).wait()

        def wait_exwrite(i, p):
            base = wid * ew + i * K
            pltpu.make_async_copy(denrows.at[p], ex_hbm.at[pl.ds(base, K)],
                                  semx[p]).wait()

        def wait_scatter(p):
            pltpu.make_async_copy(denrows.at[p], densh.at[sdx.at[p, 0]],
                                  sems[p]).wait()

        def compute_and_emit(i, p):
            base = wid * ew + i * K
            for j in range(K):
                a = rs[p, j, :] + rd[p, j, :]
                a = jnp.where(a >= 0.0, a, a * jnp.float32(0.2))
                exf = jnp.exp(a - mvec)
                denrows[p, j, :] = jnp.where(lanemask, exf, jnp.float32(0.0))
            for g in range(K // L):
                sdx[p, 0, pl.ds(g * L, L)] = dv[p, 0, pl.ds(g * L, L)]
            pltpu.async_copy(denrows.at[p], ex_hbm.at[pl.ds(base, K)], semx[p])
            pltpu.async_copy(denrows.at[p], densh.at[sdx.at[p, 0]], sems[p],
                             add=True)

        issue_idx(0, 0)
        wait_idx(0, 0)
        issue_gathers(0, 0)
        issue_idx(1, 1)

        def step(i, p, q, wait_sc):
            wait_idx(i + 1, q)
            if wait_sc:
                wait_exwrite(i - 1, q)
                wait_scatter(q)
            issue_gathers(i + 1, q)
            wait_gathers(i, p)
            compute_and_emit(i, p)
            @pl.when(i + 2 < nchunk)
            def _():
                issue_idx(i + 2, p)

        step(0, 0, 1, False)
        step(1, 1, 0, True)

        def pair_rest(t, _):
            i = 2 * t
            step(i, 0, 1, True)
            step(i + 1, 1, 0, True)
            return 0
        lax.fori_loop(1, (nchunk - 1) // 2, pair_rest, 0)
        wait_gathers(nchunk - 1, 0)
        compute_and_emit(nchunk - 1, 0)
        wait_exwrite(nchunk - 1, 0)
        wait_scatter(0)
        wait_exwrite(nchunk - 2, 1)
        wait_scatter(1)

        plsc.subcore_barrier()
        for bb in range((nblk + NS - 1) // NS):
            blk = bb * NS + s
            @pl.when(blk < nblk)
            def _():
                pltpu.sync_copy(densh.at[pl.ds(blk * zrows, zrows)], zbuf)
                pltpu.sync_copy(zbuf, den_hbm.at[c, pl.ds(blk * zrows, zrows)])

    return run(s_arr, d_arr, als, ald, m)


def _sc_pass_b(s_arr, d_arr, ex, hw, n, e, dph):
    ew = e // NW
    nchunk = ew // K
    zrows = 400
    nblk = n // zrows

    @functools.partial(
        pl.kernel,
        out_type=jax.ShapeDtypeStruct((NC, 4, n, dph), jnp.float32),
        mesh=_mesh(),
        compiler_params=pltpu.CompilerParams(use_tc_tiling_on_sc=False),
        scratch_types=[
            pltpu.VMEM((2, 1, K), jnp.int32),
            pltpu.VMEM((2, 1, K), jnp.int32),
            pltpu.VMEM((2, 1, K), jnp.int32),
            pltpu.VMEM((2, 1, K), jnp.int32),
            pltpu.VMEM((2, 1, K), jnp.float32),
            pltpu.VMEM((2, K, dph), jnp.float32),
            pltpu.VMEM((zrows, dph), jnp.float32),
            pltpu.VMEM_SHARED((n, dph), jnp.float32),
            pltpu.SemaphoreType.DMA,
            pltpu.SemaphoreType.DMA,
            pltpu.SemaphoreType.DMA,
            pltpu.SemaphoreType.DMA,
            pltpu.SemaphoreType.DMA,
            pltpu.SemaphoreType.DMA,
            pltpu.SemaphoreType.DMA,
            pltpu.SemaphoreType.DMA,
        ],
    )
    def run(s_hbm, d_hbm, ex_hbm, hw_hbm, out_hbm,
            sv, dv, sidx, sdx, exv, rows, zbuf, acc,
            semi0, semi1, semg0, semg1, seme0, seme1, sems0, sems1):
        c = lax.axis_index("c")
        s = lax.axis_index("s")
        wid = s * NC + c
        zv = jnp.zeros((L,), jnp.float32)
        semi = (semi0, semi1)
        semg = (semg0, semg1)
        seme = (seme0, seme1)
        sems = (sems0, sems1)

        def zloop(i, _):
            for q in range(dph // L):
                zbuf[i, pl.ds(q * L, L)] = zv
            return 0

        def issue_idx(i, p):
            base = wid * ew + i * K
            pltpu.async_copy(s_hbm.at[pl.ds(base, K)], sv.at[p, 0], semi[p])
            pltpu.async_copy(d_hbm.at[pl.ds(base, K)], dv.at[p, 0], semi[p])

        def wait_idx(i, p):
            base = wid * ew + i * K
            pltpu.make_async_copy(s_hbm.at[pl.ds(base, K)], sv.at[p, 0], semi[p]).wait()
            pltpu.make_async_copy(d_hbm.at[pl.ds(base, K)], dv.at[p, 0], semi[p]).wait()

        def issue_gather_ex(i, p, h):
            base = wid * ew + i * K
            for g in range(K // L):
                sidx[p, 0, pl.ds(g * L, L)] = sv[p, 0, pl.ds(g * L, L)] + jnp.int32(h * n)
            pltpu.async_copy(hw_hbm.at[sidx.at[p, 0]], rows.at[p], semg[p])
            pltpu.async_copy(ex_hbm.at[pl.ds(h * e + i * K + wid * ew, K)],
                             exv.at[p, 0], seme[p])

        def wait_gather_ex(i, p, h):
            base = wid * ew + i * K
            pltpu.make_async_copy(hw_hbm.at[sidx.at[p, 0]], rows.at[p], semg[p]).wait()
            pltpu.make_async_copy(ex_hbm.at[pl.ds(h * e + base, K)],
                                  exv.at[p, 0], seme[p]).wait()

        def wait_scatter(p):
            pltpu.make_async_copy(rows.at[p], acc.at[sdx.at[p, 0]], sems[p]).wait()

        def compute_and_scatter(i, p):
            for g in range(K // L):
                exg = exv[p, 0, pl.ds(g * L, L)]
                for j2 in range(L):
                    j = g * L + j2
                    exj = exg[j2]
                    for q in range(dph // L):
                        rows[p, j, pl.ds(q * L, L)] = rows[p, j, pl.ds(q * L, L)] * exj
            for g in range(K // L):
                sdx[p, 0, pl.ds(g * L, L)] = dv[p, 0, pl.ds(g * L, L)]
            pltpu.async_copy(rows.at[p], acc.at[sdx.at[p, 0]], sems[p], add=True)

        for h in range(4):
            lax.fori_loop(0, zrows, zloop, 0)
            for bb in range((nblk + NS - 1) // NS):
                blk = bb * NS + s
                @pl.when(blk < nblk)
                def _():
                    pltpu.sync_copy(zbuf, acc.at[pl.ds(blk * zrows, zrows)])
            plsc.subcore_barrier()

            issue_idx(0, 0)
            wait_idx(0, 0)
            issue_gather_ex(0, 0, h)
            issue_idx(1, 1)

            def step(i, p, q, wait_sc):
                wait_idx(i + 1, q)
                if wait_sc:
                    wait_scatter(q)
                issue_gather_ex(i + 1, q, h)
                wait_gather_ex(i, p, h)
                compute_and_scatter(i, p)
                @pl.when(i + 2 < nchunk)
                def _():
                    issue_idx(i + 2, p)

            step(0, 0, 1, False)
            step(1, 1, 0, True)

            def pair_rest(t, _):
                i = 2 * t
                step(i, 0, 1, True)
                step(i + 1, 1, 0, True)
                return 0
            lax.fori_loop(1, (nchunk - 1) // 2, pair_rest, 0)
            wait_gather_ex(nchunk - 1, 0, h)
            compute_and_scatter(nchunk - 1, 0)
            wait_scatter(0)
            wait_scatter(1)

            plsc.subcore_barrier()
            for bb in range((nblk + NS - 1) // NS):
                blk = bb * NS + s
                @pl.when(blk < nblk)
                def _():
                    pltpu.sync_copy(acc.at[pl.ds(blk * zrows, zrows)], zbuf)
                    pltpu.sync_copy(zbuf, out_hbm.at[c, h, pl.ds(blk * zrows, zrows)])
            plsc.subcore_barrier()

    return run(s_arr, d_arr, ex, hw)


def _layernorm(x, g, b, eps=1e-5):
    m = jnp.mean(x, axis=-1, keepdims=True)
    v = jnp.var(x, axis=-1, keepdims=True)
    return (x - m) / jnp.sqrt(v + eps) * g + b


def _gat_block(h_in, s_arr, d_arr, p, heads, dph):
    n = h_in.shape[0]
    e = s_arr.shape[0]
    hw = h_in @ p['W']
    h4 = hw.reshape(n, heads, dph)
    al_s = jnp.sum(h4 * p['a_src'][None, :, :], axis=-1)
    al_d = jnp.sum(h4 * p['a_dst'][None, :, :], axis=-1)
    ms = jnp.max(al_s, axis=0) + jnp.max(al_d, axis=0)
    m = jnp.where(ms >= 0.0, ms, 0.2 * ms)
    pad12 = jnp.zeros((n, 12), jnp.float32)
    als16 = jnp.concatenate([al_s, pad12], axis=1)
    ald16 = jnp.concatenate([al_d, pad12], axis=1)
    m16 = jnp.concatenate([m, jnp.zeros((12,), jnp.float32)])

    exr, den_p = _sc_pass_a(s_arr, d_arr, als16, ald16, m16, n, e)
    ext = jnp.transpose(exr[:, :4], (1, 0)).reshape(4 * e)
    hw_hm = jnp.transpose(h4, (1, 0, 2)).reshape(heads * n, dph)
    out_p = _sc_pass_b(s_arr, d_arr, ext, hw_hm, n, e, dph)

    alpha_self = al_s + al_d
    alpha_self = jnp.where(alpha_self >= 0.0, alpha_self, 0.2 * alpha_self)
    ex_self = jnp.exp(alpha_self - m[None, :])
    den = den_p[0, :, :4] + den_p[1, :, :4] + ex_self
    num = out_p[0] + out_p[1]
    num = jnp.transpose(num, (1, 0, 2)) + ex_self[:, :, None] * h4
    out = num / (den[:, :, None] + 1e-16)
    out = out.reshape(n, heads * dph) + p['b']
    out = _layernorm(out, p['ln_g'], p['ln_b'])
    return jax.nn.gelu(out)


def _head_kernel(gf_ref, u_ref, *refs):
    p_refs = refs[:-1]
    o_ref = refs[-1]
    (ge_W1, ge_b1, ge_g1, ge_be1, ge_W2, ge_b2, ge_g2, ge_be2,
     fu_W1, fu_b1, fu_g1, fu_be1, fu_W2, fu_b2, fu_g2, fu_be2,
     cl_W1, cl_b1, cl_W2, cl_b2) = p_refs
    u = u_ref[...]
    g = u @ ge_W1[...] + ge_b1[...][None, :]
    g = jax.nn.gelu(_layernorm(g, ge_g1[...], ge_be1[...]))
    g = g @ ge_W2[...] + ge_b2[...][None, :]
    g = jax.nn.gelu(_layernorm(g, ge_g2[...], ge_be2[...]))
    combined = jnp.concatenate([gf_ref[...], g], axis=1)
    f = combined @ fu_W1[...] + fu_b1[...][None, :]
    f = jax.nn.gelu(_layernorm(f, fu_g1[...], fu_be1[...]))
    f = f @ fu_W2[...] + fu_b2[...][None, :]
    f = jax.nn.gelu(_layernorm(f, fu_g2[...], fu_be2[...]))
    c = jax.nn.gelu(f @ cl_W1[...] + cl_b1[...][None, :])
    o_ref[...] = c @ cl_W2[...] + cl_b2[...][None, :]


def kernel(x, edge_index, batch, u, params):
    s_arr = edge_index[0]
    d_arr = edge_index[1]
    B = u.shape[0]
    h = x @ params['ne_W'] + params['ne_b']
    h = _gat_block(h, s_arr, d_arr, params['g1'], 4, 32)
    h = _gat_block(h, s_arr, d_arr, params['g2'], 4, 32)
    h = _gat_block(h, s_arr, d_arr, params['g3'], 4, 16)
    ones = jnp.ones((h.shape[0],), jnp.float32)
    cnt = jax.ops.segment_sum(ones, batch, num_segments=B)
    graph_feat = jax.ops.segment_sum(h, batch, num_segments=B) / jnp.maximum(cnt, 1.0)[:, None]

    pk = ['ge_W1', 'ge_b1', 'ge_g1', 'ge_be1', 'ge_W2', 'ge_b2', 'ge_g2', 'ge_be2',
          'fu_W1', 'fu_b1', 'fu_g1', 'fu_be1', 'fu_W2', 'fu_b2', 'fu_g2', 'fu_be2',
          'cl_W1', 'cl_b1', 'cl_W2', 'cl_b2']
    plist = [params[k] for k in pk]
    logits = pl.pallas_call(
        _head_kernel,
        out_shape=jax.ShapeDtypeStruct((B, 2), jnp.float32),
    )(graph_feat, u, *plist)
    return logits


# speedup vs baseline: 75.0054x; 1.6800x over previous
import functools

import jax
import jax.numpy as jnp
from jax import lax
from jax.experimental import pallas as pl
from jax.experimental.pallas import tpu as pltpu
from jax.experimental.pallas import tpu_sc as plsc

NC = 2
NS = 16
NW = NC * NS
L = 16
K = 80


def _mesh():
    return plsc.VectorSubcoreMesh(core_axis_name="c", subcore_axis_name="s")


def _sc_pass_a(s_arr, d_arr, als, ald, m, n, e):
    ew = e // NW
    nchunk = ew // K
    zrows = 400
    nblk = n // zrows

    @functools.partial(
        pl.kernel,
        out_type=(jax.ShapeDtypeStruct((e, 16), jnp.float32),
                  jax.ShapeDtypeStruct((NC, n, 16), jnp.float32)),
        mesh=_mesh(),
        compiler_params=pltpu.CompilerParams(use_tc_tiling_on_sc=False),
        scratch_types=[
            pltpu.VMEM((2, 1, K), jnp.int32),
            pltpu.VMEM((2, 1, K), jnp.int32),
            pltpu.VMEM((2, 1, K), jnp.int32),
            pltpu.VMEM((2, K, 16), jnp.float32),
            pltpu.VMEM((2, K, 16), jnp.float32),
            pltpu.VMEM((2, K, 16), jnp.float32),
            pltpu.VMEM((zrows, 16), jnp.float32),
            pltpu.VMEM((16,), jnp.float32),
            pltpu.VMEM_SHARED((n, 16), jnp.float32),
            pltpu.SemaphoreType.DMA,
            pltpu.SemaphoreType.DMA,
            pltpu.SemaphoreType.DMA,
            pltpu.SemaphoreType.DMA,
            pltpu.SemaphoreType.DMA,
            pltpu.SemaphoreType.DMA,
            pltpu.SemaphoreType.DMA,
            pltpu.SemaphoreType.DMA,
        ],
    )
    def run(s_hbm, d_hbm, als_hbm, ald_hbm, m_hbm, ex_hbm, den_hbm,
            sv, dv, sdx, rs, rd, denrows, zbuf, mv, densh,
            semi0, semi1, semg0, semg1, semx0, semx1, sems0, sems1):
        c = lax.axis_index("c")
        s = lax.axis_index("s")
        wid = s * NC + c
        semi = (semi0, semi1)
        semg = (semg0, semg1)
        semx = (semx0, semx1)
        sems = (sems0, sems1)
        pltpu.sync_copy(m_hbm, mv)
        zv = jnp.zeros((L,), jnp.float32)

        def zloop(i, _):
            zbuf[i, :] = zv
            return 0
        lax.fori_loop(0, zrows, zloop, 0)
        for bb in range((nblk + NS - 1) // NS):
            blk = bb * NS + s
            @pl.when(blk < nblk)
            def _():
                pltpu.sync_copy(zbuf, densh.at[pl.ds(blk * zrows, zrows)])
        plsc.subcore_barrier()

        iot = lax.iota(jnp.int32, L)
        lanemask = iot < 4
        mvec = mv[...]

        def issue_idx(i, p):
            base = wid * ew + i * K
            pltpu.async_copy(s_hbm.at[pl.ds(base, K)], sv.at[p, 0], semi[p])
            pltpu.async_copy(d_hbm.at[pl.ds(base, K)], dv.at[p, 0], semi[p])

        def wait_idx(i, p):
            base = wid * ew + i * K
            pltpu.make_async_copy(s_hbm.at[pl.ds(base, K)], sv.at[p, 0], semi[p]).wait()
            pltpu.make_async_copy(d_hbm.at[pl.ds(base, K)], dv.at[p, 0], semi[p]).wait()

        def issue_gathers(i, p):
            pltpu.async_copy(als_hbm.at[sv.at[p, 0]], rs.at[p], semg[p])
            pltpu.async_copy(ald_hbm.at[dv.at[p, 0]], rd.at[p], semg[p])

        def wait_gathers(i, p):
            pltpu.make_async_copy(als_hbm.at[sv.at[p, 0]], rs.at[p], semg[p]).wait()
            pltpu.make_async_copy(ald_hbm.at[dv.at[p, 0]], rd.at[p], semg[p]).wait()

        def wait_exwrite(i, p):
            base = wid * ew + i * K
            pltpu.make_async_copy(denrows.at[p], ex_hbm.at[pl.ds(base, K)],
                                  semx[p]).wait()

        def wait_scatter(p):
            pltpu.make_async_copy(denrows.at[p], densh.at[sdx.at[p, 0]],
                                  sems[p]).wait()

        def compute_and_emit(i, p):
            base = wid * ew + i * K
            for j in range(K):
                a = rs[p, j, :] + rd[p, j, :]
                a = jnp.where(a >= 0.0, a, a * jnp.float32(0.2))
                exf = jnp.exp(a - mvec)
                denrows[p, j, :] = jnp.where(lanemask, exf, jnp.float32(0.0))
            for g in range(K // L):
                sdx[p, 0, pl.ds(g * L, L)] = dv[p, 0, pl.ds(g * L, L)]
            pltpu.async_copy(denrows.at[p], ex_hbm.at[pl.ds(base, K)], semx[p])
            pltpu.async_copy(denrows.at[p], densh.at[sdx.at[p, 0]], sems[p],
                             add=True)

        issue_idx(0, 0)
        wait_idx(0, 0)
        issue_gathers(0, 0)
        issue_idx(1, 1)

        def step(i, p, q, wait_sc):
            wait_idx(i + 1, q)
            if wait_sc:
                wait_exwrite(i - 1, q)
                wait_scatter(q)
            issue_gathers(i + 1, q)
            wait_gathers(i, p)
            compute_and_emit(i, p)
            @pl.when(i + 2 < nchunk)
            def _():
                issue_idx(i + 2, p)

        step(0, 0, 1, False)
        step(1, 1, 0, True)

        def pair_rest(t, _):
            i = 2 * t
            step(i, 0, 1, True)
            step(i + 1, 1, 0, True)
            return 0
        lax.fori_loop(1, (nchunk - 1) // 2, pair_rest, 0)
        wait_gathers(nchunk - 1, 0)
        compute_and_emit(nchunk - 1, 0)
        wait_exwrite(nchunk - 1, 0)
        wait_scatter(0)
        wait_exwrite(nchunk - 2, 1)
        wait_scatter(1)

        plsc.subcore_barrier()
        for bb in range((nblk + NS - 1) // NS):
            blk = bb * NS + s
            @pl.when(blk < nblk)
            def _():
                pltpu.sync_copy(densh.at[pl.ds(blk * zrows, zrows)], zbuf)
                pltpu.sync_copy(zbuf, den_hbm.at[c, pl.ds(blk * zrows, zrows)])

    return run(s_arr, d_arr, als, ald, m)


def _sc_pass_b(s_arr, d_arr, ex, hw, n, e, dph):
    ew = e // NW
    nchunk = ew // K
    zrows = 400
    nblk = n // zrows

    @functools.partial(
        pl.kernel,
        out_type=jax.ShapeDtypeStruct((NC, 4, n, dph), jnp.float32),
        mesh=_mesh(),
        compiler_params=pltpu.CompilerParams(use_tc_tiling_on_sc=False),
        scratch_types=[
            pltpu.VMEM((2, 1, K), jnp.int32),
            pltpu.VMEM((2, 1, K), jnp.int32),
            pltpu.VMEM((2, 1, K), jnp.int32),
            pltpu.VMEM((2, 1, K), jnp.int32),
            pltpu.VMEM((2, K, 16), jnp.float32),
            pltpu.VMEM((2, K, dph), jnp.float32),
            pltpu.VMEM((zrows, dph), jnp.float32),
            pltpu.VMEM_SHARED((n, dph), jnp.float32),
            pltpu.SemaphoreType.DMA,
            pltpu.SemaphoreType.DMA,
            pltpu.SemaphoreType.DMA,
            pltpu.SemaphoreType.DMA,
            pltpu.SemaphoreType.DMA,
            pltpu.SemaphoreType.DMA,
            pltpu.SemaphoreType.DMA,
            pltpu.SemaphoreType.DMA,
        ],
    )
    def run(s_hbm, d_hbm, ex_hbm, hw_hbm, out_hbm,
            sv, dv, sidx, sdx, exv, rows, zbuf, acc,
            semi0, semi1, semg0, semg1, seme0, seme1, sems0, sems1):
        c = lax.axis_index("c")
        s = lax.axis_index("s")
        wid = s * NC + c
        zv = jnp.zeros((L,), jnp.float32)
        semi = (semi0, semi1)
        semg = (semg0, semg1)
        seme = (seme0, seme1)
        sems = (sems0, sems1)

        def zloop(i, _):
            for q in range(dph // L):
                zbuf[i, pl.ds(q * L, L)] = zv
            return 0

        def issue_idx(i, p):
            base = wid * ew + i * K
            pltpu.async_copy(s_hbm.at[pl.ds(base, K)], sv.at[p, 0], semi[p])
            pltpu.async_copy(d_hbm.at[pl.ds(base, K)], dv.at[p, 0], semi[p])

        def wait_idx(i, p):
            base = wid * ew + i * K
            pltpu.make_async_copy(s_hbm.at[pl.ds(base, K)], sv.at[p, 0], semi[p]).wait()
            pltpu.make_async_copy(d_hbm.at[pl.ds(base, K)], dv.at[p, 0], semi[p]).wait()

        def issue_gather_ex(i, p, h):
            base = wid * ew + i * K
            for g in range(K // L):
                sidx[p, 0, pl.ds(g * L, L)] = sv[p, 0, pl.ds(g * L, L)] + jnp.int32(h * n)
            pltpu.async_copy(hw_hbm.at[sidx.at[p, 0]], rows.at[p], semg[p])
            pltpu.async_copy(ex_hbm.at[pl.ds(base, K)], exv.at[p], seme[p])

        def wait_gather_ex(i, p, h):
            base = wid * ew + i * K
            pltpu.make_async_copy(hw_hbm.at[sidx.at[p, 0]], rows.at[p], semg[p]).wait()
            pltpu.make_async_copy(ex_hbm.at[pl.ds(base, K)], exv.at[p], seme[p]).wait()

        def wait_scatter(p):
            pltpu.make_async_copy(rows.at[p], acc.at[sdx.at[p, 0]], sems[p]).wait()

        def compute_and_scatter(i, p):
            for j in range(K):
                exj = exv[p, j, :][h]
                for q in range(dph // L):
                    rows[p, j, pl.ds(q * L, L)] = rows[p, j, pl.ds(q * L, L)] * exj
            for g in range(K // L):
                sdx[p, 0, pl.ds(g * L, L)] = dv[p, 0, pl.ds(g * L, L)]
            pltpu.async_copy(rows.at[p], acc.at[sdx.at[p, 0]], sems[p], add=True)

        for h in range(4):
            lax.fori_loop(0, zrows, zloop, 0)
            for bb in range((nblk + NS - 1) // NS):
                blk = bb * NS + s
                @pl.when(blk < nblk)
                def _():
                    pltpu.sync_copy(zbuf, acc.at[pl.ds(blk * zrows, zrows)])
            plsc.subcore_barrier()

            issue_idx(0, 0)
            wait_idx(0, 0)
            issue_gather_ex(0, 0, h)
            issue_idx(1, 1)

            def step(i, p, q, wait_sc):
                wait_idx(i + 1, q)
                if wait_sc:
                    wait_scatter(q)
                issue_gather_ex(i + 1, q, h)
                wait_gather_ex(i, p, h)
                compute_and_scatter(i, p)
                @pl.when(i + 2 < nchunk)
                def _():
                    issue_idx(i + 2, p)

            step(0, 0, 1, False)
            step(1, 1, 0, True)

            def pair_rest(t, _):
                i = 2 * t
                step(i, 0, 1, True)
                step(i + 1, 1, 0, True)
                return 0
            lax.fori_loop(1, (nchunk - 1) // 2, pair_rest, 0)
            wait_gather_ex(nchunk - 1, 0, h)
            compute_and_scatter(nchunk - 1, 0)
            wait_scatter(0)
            wait_scatter(1)

            plsc.subcore_barrier()
            for bb in range((nblk + NS - 1) // NS):
                blk = bb * NS + s
                @pl.when(blk < nblk)
                def _():
                    pltpu.sync_copy(acc.at[pl.ds(blk * zrows, zrows)], zbuf)
                    pltpu.sync_copy(zbuf, out_hbm.at[c, h, pl.ds(blk * zrows, zrows)])
            plsc.subcore_barrier()

    return run(s_arr, d_arr, ex, hw)


def _layernorm(x, g, b, eps=1e-5):
    m = jnp.mean(x, axis=-1, keepdims=True)
    v = jnp.var(x, axis=-1, keepdims=True)
    return (x - m) / jnp.sqrt(v + eps) * g + b


NEG = -3e38
R = 2000


def _al_proj(a):
    heads, dph = a.shape
    return (jnp.eye(heads, dtype=a.dtype)[:, None, :] * a[:, :, None]).reshape(
        heads * dph, heads)


def _accum_max(ref, i, als, ald):
    bm0 = jnp.concatenate([jnp.max(als, axis=0, keepdims=True),
                           jnp.full((1, 124), NEG, jnp.float32)], axis=1)
    bm1 = jnp.concatenate([jnp.max(ald, axis=0, keepdims=True),
                           jnp.full((1, 124), NEG, jnp.float32)], axis=1)
    m_blk = jnp.concatenate([bm0, bm1, jnp.full((6, 128), NEG, jnp.float32)], 0)

    @pl.when(i == 0)
    def _():
        ref[...] = jnp.full((8, 128), NEG, jnp.float32)
    ref[...] = jnp.maximum(ref[...], m_blk)


def _emit_hw(hw_ref, als_ref, ald_ref, hw, als, ald, dph):
    for h in range(4):
        hw_ref[h] = hw[:, h * dph:(h + 1) * dph]
    z12 = jnp.zeros((hw.shape[0], 12), jnp.float32)
    als_ref[...] = jnp.concatenate([als, z12], axis=1)
    ald_ref[...] = jnp.concatenate([ald, z12], axis=1)


def _tc_pre_kernel(x_ref, new_ref, neb_ref, w1_ref, ps_ref, pd_ref,
                   hw_ref, als_ref, ald_ref, m_ref):
    i = pl.program_id(0)
    h0 = x_ref[...] @ new_ref[...] + neb_ref[...]
    hw = h0 @ w1_ref[...]
    als = hw @ ps_ref[...]
    ald = hw @ pd_ref[...]
    _emit_hw(hw_ref, als_ref, ald_ref, hw, als, ald, 32)
    _accum_max(m_ref, i, als, ald)


def _combine(outp, denp, hwp, als16, ald16, m16, bias, lng, lnb, dph):
    m4 = m16[:, 0:4]
    aself = als16[:, 0:4] + ald16[:, 0:4]
    aself = jnp.where(aself >= 0.0, aself, 0.2 * aself)
    exs = jnp.exp(aself - m4)
    den = denp[0, :, 0:4] + denp[1, :, 0:4] + exs
    outs = []
    for h in range(4):
        numh = outp[0, h] + outp[1, h] + exs[:, h:h + 1] * hwp[h]
        outs.append(numh / (den[:, h:h + 1] + 1e-16))
    hcat = jnp.concatenate(outs, axis=1) + bias
    hcat = _layernorm(hcat, lng, lnb)
    return jax.nn.gelu(hcat)


def _tc_mid_kernel(outp_ref, denp_ref, hwp_ref, alsp_ref, aldp_ref, m16_ref,
                   b_ref, g_ref, be_ref, wn_ref, ps_ref, pd_ref,
                   hw_ref, als_ref, ald_ref, m_ref, *, dph_in, dph_out):
    i = pl.program_id(0)
    hn = _combine(outp_ref[...], denp_ref[...], hwp_ref[...], alsp_ref[...],
                  aldp_ref[...], m16_ref[...], b_ref[...], g_ref[...],
                  be_ref[...], dph_in)
    hw = hn @ wn_ref[...]
    als = hw @ ps_ref[...]
    ald = hw @ pd_ref[...]
    _emit_hw(hw_ref, als_ref, ald_ref, hw, als, ald, dph_out)
    _accum_max(m_ref, i, als, ald)


def _tc_post_kernel(outp_ref, denp_ref, hwp_ref, alsp_ref, aldp_ref, m16_ref,
                    b_ref, g_ref, be_ref, oh_ref, pool_ref):
    i = pl.program_id(0)
    h3 = _combine(outp_ref[...], denp_ref[...], hwp_ref[...], alsp_ref[...],
                  aldp_ref[...], m16_ref[...], b_ref[...], g_ref[...],
                  be_ref[...], 16)
    ext = jnp.concatenate([h3, jnp.ones((h3.shape[0], 64), jnp.float32)], 1)
    part = jax.lax.dot_general(oh_ref[...], ext, (((0,), (0,)), ((), ())))

    @pl.when(i == 0)
    def _():
        pool_ref[...] = jnp.zeros((32, 128), jnp.float32)
    pool_ref[...] = pool_ref[...] + part


def _head_kernel(pool_ref, u_ref, *refs):
    p_refs = refs[:-1]
    o_ref = refs[-1]
    (ge_W1, ge_b1, ge_g1, ge_be1, ge_W2, ge_b2, ge_g2, ge_be2,
     fu_W1, fu_b1, fu_g1, fu_be1, fu_W2, fu_b2, fu_g2, fu_be2,
     cl_W1, cl_b1, cl_W2, cl_b2) = p_refs
    pool = pool_ref[...]
    gf = pool[:, 0:64] / jnp.maximum(pool[:, 64:65], 1.0)
    u = u_ref[...]
    g = u @ ge_W1[...] + ge_b1[...][None, :]
    g = jax.nn.gelu(_layernorm(g, ge_g1[...], ge_be1[...]))
    g = g @ ge_W2[...] + ge_b2[...][None, :]
    g = jax.nn.gelu(_layernorm(g, ge_g2[...], ge_be2[...]))
    combined = jnp.concatenate([gf, g], axis=1)
    f = combined @ fu_W1[...] + fu_b1[...][None, :]
    f = jax.nn.gelu(_layernorm(f, fu_g1[...], fu_be1[...]))
    f = f @ fu_W2[...] + fu_b2[...][None, :]
    f = jax.nn.gelu(_layernorm(f, fu_g2[...], fu_be2[...]))
    c = jax.nn.gelu(f @ cl_W1[...] + cl_b1[...][None, :])
    o_ref[...] = c @ cl_W2[...] + cl_b2[...][None, :]


def _full(shape):
    return pl.BlockSpec(shape, lambda i: tuple(0 for _ in shape))


def _m16_of(mout):
    m4 = mout[0, 0:4] + mout[1, 0:4]
    m4 = jnp.where(m4 >= 0.0, m4, 0.2 * m4)
    return jnp.concatenate([m4, jnp.zeros((12,), jnp.float32)])


def kernel(x, edge_index, batch, u, params):
    s_arr = edge_index[0]
    d_arr = edge_index[1]
    n = x.shape[0]
    e = s_arr.shape[0]
    B = u.shape[0]
    nb = n // R
    f32 = jnp.float32
    p1, p2, p3 = params['g1'], params['g2'], params['g3']

    hw1, als1, ald1, mo1 = pl.pallas_call(
        _tc_pre_kernel,
        grid=(nb,),
        in_specs=[
            pl.BlockSpec((R, 16), lambda i: (i, 0)),
            _full((16, 64)), _full((1, 64)), _full((64, 128)),
            _full((128, 4)), _full((128, 4)),
        ],
        out_specs=[
            pl.BlockSpec((4, R, 32), lambda i: (0, i, 0)),
            pl.BlockSpec((R, 16), lambda i: (i, 0)),
            pl.BlockSpec((R, 16), lambda i: (i, 0)),
            _full((8, 128)),
        ],
        out_shape=[
            jax.ShapeDtypeStruct((4, n, 32), f32),
            jax.ShapeDtypeStruct((n, 16), f32),
            jax.ShapeDtypeStruct((n, 16), f32),
            jax.ShapeDtypeStruct((8, 128), f32),
        ],
    )(x, params['ne_W'], params['ne_b'].reshape(1, 64), p1['W'],
      _al_proj(p1['a_src']), _al_proj(p1['a_dst']))

    def sc_layer(hw, als16, ald16, mo, dph):
        m16 = _m16_of(mo)
        exr, den_p = _sc_pass_a(s_arr, d_arr, als16, ald16, m16, n, e)
        out_p = _sc_pass_b(s_arr, d_arr, exr, hw.reshape(4 * n, dph), n, e, dph)
        return m16, den_p, out_p

    m16_1, den1, out1 = sc_layer(hw1, als1, ald1, mo1, 32)

    def mid_call(dph_in, dph_out, out_p, den_p, hw, als16, ald16, m16,
                 pblk, wn, psn, pdn):
        return pl.pallas_call(
            functools.partial(_tc_mid_kernel, dph_in=dph_in, dph_out=dph_out),
            grid=(nb,),
            in_specs=[
                pl.BlockSpec((NC, 4, R, dph_in), lambda i: (0, 0, i, 0)),
                pl.BlockSpec((NC, R, 16), lambda i: (0, i, 0)),
                pl.BlockSpec((4, R, dph_in), lambda i: (0, i, 0)),
                pl.BlockSpec((R, 16), lambda i: (i, 0)),
                pl.BlockSpec((R, 16), lambda i: (i, 0)),
                _full((1, 16)),
                _full((1, 4 * dph_in)), _full((1, 4 * dph_in)), _full((1, 4 * dph_in)),
                _full((4 * dph_in, 4 * dph_out)),
                _full((4 * dph_out, 4)), _full((4 * dph_out, 4)),
            ],
            out_specs=[
                pl.BlockSpec((4, R, dph_out), lambda i: (0, i, 0)),
                pl.BlockSpec((R, 16), lambda i: (i, 0)),
                pl.BlockSpec((R, 16), lambda i: (i, 0)),
                _full((8, 128)),
            ],
            out_shape=[
                jax.ShapeDtypeStruct((4, n, dph_out), f32),
                jax.ShapeDtypeStruct((n, 16), f32),
                jax.ShapeDtypeStruct((n, 16), f32),
                jax.ShapeDtypeStruct((8, 128), f32),
            ],
        )(out_p, den_p, hw, als16, ald16, m16.reshape(1, 16),
          pblk['b'].reshape(1, -1), pblk['ln_g'].reshape(1, -1),
          pblk['ln_b'].reshape(1, -1), wn, psn, pdn)

    hw2, als2, ald2, mo2 = mid_call(32, 32, out1, den1, hw1, als1, ald1, m16_1,
                                    p1, p2['W'], _al_proj(p2['a_src']),
                                    _al_proj(p2['a_dst']))
    m16_2, den2, out2 = sc_layer(hw2, als2, ald2, mo2, 32)

    hw3, als3, ald3, mo3 = mid_call(32, 16, out2, den2, hw2, als2, ald2, m16_2,
                                    p2, p3['W'], _al_proj(p3['a_src']),
                                    _al_proj(p3['a_dst']))
    m16_3, den3, out3 = sc_layer(hw3, als3, ald3, mo3, 16)

    onehot = (batch[:, None] == jnp.arange(B, dtype=batch.dtype)[None, :]
              ).astype(f32)
    pooled = pl.pallas_call(
        _tc_post_kernel,
        grid=(nb,),
        in_specs=[
            pl.BlockSpec((NC, 4, R, 16), lambda i: (0, 0, i, 0)),
            pl.BlockSpec((NC, R, 16), lambda i: (0, i, 0)),
            pl.BlockSpec((4, R, 16), lambda i: (0, i, 0)),
            pl.BlockSpec((R, 16), lambda i: (i, 0)),
            pl.BlockSpec((R, 16), lambda i: (i, 0)),
            _full((1, 16)),
            _full((1, 64)), _full((1, 64)), _full((1, 64)),
            pl.BlockSpec((R, 32), lambda i: (i, 0)),
        ],
        out_specs=[_full((32, 128))],
        out_shape=[jax.ShapeDtypeStruct((32, 128), f32)],
    )(out3, den3, hw3, als3, ald3, m16_3.reshape(1, 16),
      p3['b'].reshape(1, -1), p3['ln_g'].reshape(1, -1),
      p3['ln_b'].reshape(1, -1), onehot)[0]

    pk = ['ge_W1', 'ge_b1', 'ge_g1', 'ge_be1', 'ge_W2', 'ge_b2', 'ge_g2', 'ge_be2',
          'fu_W1', 'fu_b1', 'fu_g1', 'fu_be1', 'fu_W2', 'fu_b2', 'fu_g2', 'fu_be2',
          'cl_W1', 'cl_b1', 'cl_W2', 'cl_b2']
    plist = [params[k] for k in pk]
    logits = pl.pallas_call(
        _head_kernel,
        out_shape=jax.ShapeDtypeStruct((B, 2), f32),
    )(pooled, u, *plist)
    return logits


# speedup vs baseline: 75.0916x; 1.0011x over previous
import functools

import jax
import jax.numpy as jnp
from jax import lax
from jax.experimental import pallas as pl
from jax.experimental.pallas import tpu as pltpu
from jax.experimental.pallas import tpu_sc as plsc

NC = 2
NS = 16
NW = NC * NS
L = 16
K = 80


def _mesh():
    return plsc.VectorSubcoreMesh(core_axis_name="c", subcore_axis_name="s")


def _sc_pass_a(s_arr, d_arr, als, ald, m, n, e):
    ew = e // NW
    nchunk = ew // K
    zrows = 400
    nblk = n // zrows

    @functools.partial(
        pl.kernel,
        out_type=(jax.ShapeDtypeStruct((e, 16), jnp.float32),
                  jax.ShapeDtypeStruct((NC, n, 16), jnp.float32)),
        mesh=_mesh(),
        compiler_params=pltpu.CompilerParams(use_tc_tiling_on_sc=False),
        scratch_types=[
            pltpu.VMEM((2, 1, K), jnp.int32),
            pltpu.VMEM((2, 1, K), jnp.int32),
            pltpu.VMEM((2, 1, K), jnp.int32),
            pltpu.VMEM((2, K, 16), jnp.float32),
            pltpu.VMEM((2, K, 16), jnp.float32),
            pltpu.VMEM((2, K, 16), jnp.float32),
            pltpu.VMEM((zrows, 16), jnp.float32),
            pltpu.VMEM((16,), jnp.float32),
            pltpu.VMEM_SHARED((n, 16), jnp.float32),
            pltpu.SemaphoreType.DMA,
            pltpu.SemaphoreType.DMA,
            pltpu.SemaphoreType.DMA,
            pltpu.SemaphoreType.DMA,
            pltpu.SemaphoreType.DMA,
            pltpu.SemaphoreType.DMA,
            pltpu.SemaphoreType.DMA,
            pltpu.SemaphoreType.DMA,
        ],
    )
    def run(s_hbm, d_hbm, als_hbm, ald_hbm, m_hbm, ex_hbm, den_hbm,
            sv, dv, sdx, rs, rd, denrows, zbuf, mv, densh,
            semi0, semi1, semg0, semg1, semx0, semx1, sems0, sems1):
        c = lax.axis_index("c")
        s = lax.axis_index("s")
        wid = s * NC + c
        semi = (semi0, semi1)
        semg = (semg0, semg1)
        semx = (semx0, semx1)
        sems = (sems0, sems1)
        pltpu.sync_copy(m_hbm, mv)
        zv = jnp.zeros((L,), jnp.float32)

        def zloop(i, _):
            zbuf[i, :] = zv
            return 0
        lax.fori_loop(0, zrows, zloop, 0)
        for bb in range((nblk + NS - 1) // NS):
            blk = bb * NS + s
            @pl.when(blk < nblk)
            def _():
                pltpu.sync_copy(zbuf, densh.at[pl.ds(blk * zrows, zrows)])
        plsc.subcore_barrier()

        iot = lax.iota(jnp.int32, L)
        lanemask = iot < 4
        mvec = mv[...]

        def issue_idx(i, p):
            base = wid * ew + i * K
            pltpu.async_copy(s_hbm.at[pl.ds(base, K)], sv.at[p, 0], semi[p])
            pltpu.async_copy(d_hbm.at[pl.ds(base, K)], dv.at[p, 0], semi[p])

        def wait_idx(i, p):
            base = wid * ew + i * K
            pltpu.make_async_copy(s_hbm.at[pl.ds(base, K)], sv.at[p, 0], semi[p]).wait()
            pltpu.make_async_copy(d_hbm.at[pl.ds(base, K)], dv.at[p, 0], semi[p]).wait()

        def issue_gathers(i, p):
            pltpu.async_copy(als_hbm.at[sv.at[p, 0]], rs.at[p], semg[p])
            pltpu.async_copy(ald_hbm.at[dv.at[p, 0]], rd.at[p], semg[p])

        def wait_gathers(i, p):
            pltpu.make_async_copy(als_hbm.at[sv.at[p, 0]], rs.at[p], semg[p]).wait()
            pltpu.make_async_copy(ald_hbm.at[dv.at[p, 0]], rd.at[p], semg[p]).wait()

        def wait_exwrite(i, p):
            base = wid * ew + i * K
            pltpu.make_async_copy(denrows.at[p], ex_hbm.at[pl.ds(base, K)],
                                  semx[p]).wait()

        def wait_scatter(p):
            pltpu.make_async_copy(denrows.at[p], densh.at[sdx.at[p, 0]],
                                  sems[p]).wait()

        def compute_and_emit(i, p):
            base = wid * ew + i * K
            for j in range(K):
                a = rs[p, j, :] + rd[p, j, :]
                a = jnp.where(a >= 0.0, a, a * jnp.float32(0.2))
                exf = jnp.exp(a - mvec)
                denrows[p, j, :] = jnp.where(lanemask, exf, jnp.float32(0.0))
            for g in range(K // L):
                sdx[p, 0, pl.ds(g * L, L)] = dv[p, 0, pl.ds(g * L, L)]
            pltpu.async_copy(denrows.at[p], ex_hbm.at[pl.ds(base, K)], semx[p])
            pltpu.async_copy(denrows.at[p], densh.at[sdx.at[p, 0]], sems[p],
                             add=True)

        issue_idx(0, 0)
        wait_idx(0, 0)
        issue_gathers(0, 0)
        issue_idx(1, 1)

        def step(i, p, q, wait_sc):
            wait_idx(i + 1, q)
            if wait_sc:
                wait_exwrite(i - 1, q)
                wait_scatter(q)
            issue_gathers(i + 1, q)
            wait_gathers(i, p)
            compute_and_emit(i, p)
            @pl.when(i + 2 < nchunk)
            def _():
                issue_idx(i + 2, p)

        step(0, 0, 1, False)
        step(1, 1, 0, True)

        def pair_rest(t, _):
            i = 2 * t
            step(i, 0, 1, True)
            step(i + 1, 1, 0, True)
            return 0
        lax.fori_loop(1, (nchunk - 1) // 2, pair_rest, 0)
        wait_gathers(nchunk - 1, 0)
        compute_and_emit(nchunk - 1, 0)
        wait_exwrite(nchunk - 1, 0)
        wait_scatter(0)
        wait_exwrite(nchunk - 2, 1)
        wait_scatter(1)

        plsc.subcore_barrier()
        for bb in range((nblk + NS - 1) // NS):
            blk = bb * NS + s
            @pl.when(blk < nblk)
            def _():
                pltpu.sync_copy(densh.at[pl.ds(blk * zrows, zrows)], zbuf)
                pltpu.sync_copy(zbuf, den_hbm.at[c, pl.ds(blk * zrows, zrows)])

    return run(s_arr, d_arr, als, ald, m)


def _sc_pass_b(s_arr, d_arr, ex, hw, n, e, dph):
    ew = e // NW
    nchunk = ew // K
    zrows = 400
    nblk = n // zrows

    @functools.partial(
        pl.kernel,
        out_type=jax.ShapeDtypeStruct((NC, 4, n, dph), jnp.float32),
        mesh=_mesh(),
        compiler_params=pltpu.CompilerParams(use_tc_tiling_on_sc=False),
        scratch_types=[
            pltpu.VMEM((2, 1, K), jnp.int32),
            pltpu.VMEM((2, 1, K), jnp.int32),
            pltpu.VMEM((2, 1, K), jnp.int32),
            pltpu.VMEM((2, 1, K), jnp.int32),
            pltpu.VMEM((2, K, 16), jnp.float32),
            pltpu.VMEM((2, K, dph), jnp.float32),
            pltpu.VMEM((zrows, dph), jnp.float32),
            pltpu.VMEM_SHARED((n, dph), jnp.float32),
            pltpu.SemaphoreType.DMA,
            pltpu.SemaphoreType.DMA,
            pltpu.SemaphoreType.DMA,
            pltpu.SemaphoreType.DMA,
            pltpu.SemaphoreType.DMA,
            pltpu.SemaphoreType.DMA,
            pltpu.SemaphoreType.DMA,
            pltpu.SemaphoreType.DMA,
        ],
    )
    def run(s_hbm, d_hbm, ex_hbm, hw_hbm, out_hbm,
            sv, dv, sidx, sdx, exv, rows, zbuf, acc,
            semi0, semi1, semg0, semg1, seme0, seme1, sems0, sems1):
        c = lax.axis_index("c")
        s = lax.axis_index("s")
        wid = s * NC + c
        zv = jnp.zeros((L,), jnp.float32)
        semi = (semi0, semi1)
        semg = (semg0, semg1)
        seme = (seme0, seme1)
        sems = (sems0, sems1)

        def zloop(i, _):
            for q in range(dph // L):
                zbuf[i, pl.ds(q * L, L)] = zv
            return 0

        def issue_idx(i, p):
            base = wid * ew + i * K
            pltpu.async_copy(s_hbm.at[pl.ds(base, K)], sv.at[p, 0], semi[p])
            pltpu.async_copy(d_hbm.at[pl.ds(base, K)], dv.at[p, 0], semi[p])

        def wait_idx(i, p):
            base = wid * ew + i * K
            pltpu.make_async_copy(s_hbm.at[pl.ds(base, K)], sv.at[p, 0], semi[p]).wait()
            pltpu.make_async_copy(d_hbm.at[pl.ds(base, K)], dv.at[p, 0], semi[p]).wait()

        def issue_gather_ex(i, p, h):
            base = wid * ew + i * K
            for g in range(K // L):
                sidx[p, 0, pl.ds(g * L, L)] = (
                    sv[p, 0, pl.ds(g * L, L)] + jnp.int32(h * n))
            pltpu.async_copy(hw_hbm.at[sidx.at[p, 0]], rows.at[p], semg[p])
            pltpu.async_copy(ex_hbm.at[pl.ds(base, K)], exv.at[p], seme[p])

        def wait_gather_ex(i, p, h):
            base = wid * ew + i * K
            pltpu.make_async_copy(hw_hbm.at[sidx.at[p, 0]], rows.at[p], semg[p]).wait()
            pltpu.make_async_copy(ex_hbm.at[pl.ds(base, K)], exv.at[p], seme[p]).wait()

        def wait_scatter(p):
            pltpu.make_async_copy(rows.at[p], acc.at[sdx.at[p, 0]], sems[p]).wait()

        def compute_and_scatter(i, p, h):
            for j in range(K):
                exj = exv[p, j, :][h]
                for q in range(dph // L):
                    rows[p, j, pl.ds(q * L, L)] = rows[p, j, pl.ds(q * L, L)] * exj
            for g in range(K // L):
                sdx[p, 0, pl.ds(g * L, L)] = dv[p, 0, pl.ds(g * L, L)]
            pltpu.async_copy(rows.at[p], acc.at[sdx.at[p, 0]], sems[p], add=True)

        for h in range(4):
            lax.fori_loop(0, zrows, zloop, 0)
            for bb in range((nblk + NS - 1) // NS):
                blk = bb * NS + s
                @pl.when(blk < nblk)
                def _():
                    pltpu.sync_copy(zbuf, acc.at[pl.ds(blk * zrows, zrows)])
            plsc.subcore_barrier()

            issue_idx(0, 0)
            wait_idx(0, 0)
            issue_gather_ex(0, 0, h)
            issue_idx(1, 1)

            def step(i, p, q, wait_sc):
                wait_idx(i + 1, q)
                if wait_sc:
                    wait_scatter(q)
                issue_gather_ex(i + 1, q, h)
                wait_gather_ex(i, p, h)
                compute_and_scatter(i, p, h)
                @pl.when(i + 2 < nchunk)
                def _():
                    issue_idx(i + 2, p)

            step(0, 0, 1, False)
            step(1, 1, 0, True)

            def pair_rest(t, _):
                i = 2 * t
                step(i, 0, 1, True)
                step(i + 1, 1, 0, True)
                return 0
            lax.fori_loop(1, (nchunk - 1) // 2, pair_rest, 0)
            wait_gather_ex(nchunk - 1, 0, h)
            compute_and_scatter(nchunk - 1, 0, h)
            wait_scatter(0)
            wait_scatter(1)

            plsc.subcore_barrier()
            for bb in range((nblk + NS - 1) // NS):
                blk = bb * NS + s
                @pl.when(blk < nblk)
                def _():
                    pltpu.sync_copy(acc.at[pl.ds(blk * zrows, zrows)], zbuf)
                    pltpu.sync_copy(zbuf, out_hbm.at[c, h, pl.ds(blk * zrows, zrows)])
            plsc.subcore_barrier()

    return run(s_arr, d_arr, ex, hw)


def _layernorm(x, g, b, eps=1e-5):
    m = jnp.mean(x, axis=-1, keepdims=True)
    v = jnp.var(x, axis=-1, keepdims=True)
    return (x - m) / jnp.sqrt(v + eps) * g + b


NEG = -3e38
R = 2000


def _al_proj(a):
    heads, dph = a.shape
    return (jnp.eye(heads, dtype=a.dtype)[:, None, :] * a[:, :, None]).reshape(
        heads * dph, heads)


def _accum_max(ref, i, als, ald):
    bm0 = jnp.concatenate([jnp.max(als, axis=0, keepdims=True),
                           jnp.full((1, 124), NEG, jnp.float32)], axis=1)
    bm1 = jnp.concatenate([jnp.max(ald, axis=0, keepdims=True),
                           jnp.full((1, 124), NEG, jnp.float32)], axis=1)
    m_blk = jnp.concatenate([bm0, bm1, jnp.full((6, 128), NEG, jnp.float32)], 0)

    @pl.when(i == 0)
    def _():
        ref[...] = jnp.full((8, 128), NEG, jnp.float32)
    ref[...] = jnp.maximum(ref[...], m_blk)


def _emit_hw(hw_ref, als_ref, ald_ref, hw, als, ald, dph):
    for h in range(4):
        hw_ref[h] = hw[:, h * dph:(h + 1) * dph]
    z12 = jnp.zeros((hw.shape[0], 12), jnp.float32)
    als_ref[...] = jnp.concatenate([als, z12], axis=1)
    ald_ref[...] = jnp.concatenate([ald, z12], axis=1)


def _tc_pre_kernel(x_ref, new_ref, neb_ref, w1_ref, ps_ref, pd_ref,
                   hw_ref, als_ref, ald_ref, m_ref):
    i = pl.program_id(0)
    h0 = x_ref[...] @ new_ref[...] + neb_ref[...]
    hw = h0 @ w1_ref[...]
    als = hw @ ps_ref[...]
    ald = hw @ pd_ref[...]
    _emit_hw(hw_ref, als_ref, ald_ref, hw, als, ald, 32)
    _accum_max(m_ref, i, als, ald)


def _combine(outp, denp, hwp, als16, ald16, m16, bias, lng, lnb, dph):
    m4 = m16[:, 0:4]
    aself = als16[:, 0:4] + ald16[:, 0:4]
    aself = jnp.where(aself >= 0.0, aself, 0.2 * aself)
    exs = jnp.exp(aself - m4)
    den = denp[0, :, 0:4] + denp[1, :, 0:4] + exs
    outs = []
    for h in range(4):
        numh = outp[0, h] + outp[1, h] + exs[:, h:h + 1] * hwp[h]
        outs.append(numh / (den[:, h:h + 1] + 1e-16))
    hcat = jnp.concatenate(outs, axis=1) + bias
    hcat = _layernorm(hcat, lng, lnb)
    return jax.nn.gelu(hcat)


def _tc_mid_kernel(outp_ref, denp_ref, hwp_ref, alsp_ref, aldp_ref, m16_ref,
                   b_ref, g_ref, be_ref, wn_ref, ps_ref, pd_ref,
                   hw_ref, als_ref, ald_ref, m_ref, *, dph_in, dph_out):
    i = pl.program_id(0)
    hn = _combine(outp_ref[...], denp_ref[...], hwp_ref[...], alsp_ref[...],
                  aldp_ref[...], m16_ref[...], b_ref[...], g_ref[...],
                  be_ref[...], dph_in)
    hw = hn @ wn_ref[...]
    als = hw @ ps_ref[...]
    ald = hw @ pd_ref[...]
    _emit_hw(hw_ref, als_ref, ald_ref, hw, als, ald, dph_out)
    _accum_max(m_ref, i, als, ald)


def _tc_post_kernel(outp_ref, denp_ref, hwp_ref, alsp_ref, aldp_ref, m16_ref,
                    b_ref, g_ref, be_ref, oh_ref, pool_ref):
    i = pl.program_id(0)
    h3 = _combine(outp_ref[...], denp_ref[...], hwp_ref[...], alsp_ref[...],
                  aldp_ref[...], m16_ref[...], b_ref[...], g_ref[...],
                  be_ref[...], 16)
    ext = jnp.concatenate([h3, jnp.ones((h3.shape[0], 64), jnp.float32)], 1)
    part = jax.lax.dot_general(oh_ref[...], ext, (((0,), (0,)), ((), ())))

    @pl.when(i == 0)
    def _():
        pool_ref[...] = jnp.zeros((32, 128), jnp.float32)
    pool_ref[...] = pool_ref[...] + part


def _head_kernel(pool_ref, u_ref, *refs):
    p_refs = refs[:-1]
    o_ref = refs[-1]
    (ge_W1, ge_b1, ge_g1, ge_be1, ge_W2, ge_b2, ge_g2, ge_be2,
     fu_W1, fu_b1, fu_g1, fu_be1, fu_W2, fu_b2, fu_g2, fu_be2,
     cl_W1, cl_b1, cl_W2, cl_b2) = p_refs
    pool = pool_ref[...]
    gf = pool[:, 0:64] / jnp.maximum(pool[:, 64:65], 1.0)
    u = u_ref[...]
    g = u @ ge_W1[...] + ge_b1[...][None, :]
    g = jax.nn.gelu(_layernorm(g, ge_g1[...], ge_be1[...]))
    g = g @ ge_W2[...] + ge_b2[...][None, :]
    g = jax.nn.gelu(_layernorm(g, ge_g2[...], ge_be2[...]))
    combined = jnp.concatenate([gf, g], axis=1)
    f = combined @ fu_W1[...] + fu_b1[...][None, :]
    f = jax.nn.gelu(_layernorm(f, fu_g1[...], fu_be1[...]))
    f = f @ fu_W2[...] + fu_b2[...][None, :]
    f = jax.nn.gelu(_layernorm(f, fu_g2[...], fu_be2[...]))
    c = jax.nn.gelu(f @ cl_W1[...] + cl_b1[...][None, :])
    o_ref[...] = c @ cl_W2[...] + cl_b2[...][None, :]


def _full(shape):
    return pl.BlockSpec(shape, lambda i: tuple(0 for _ in shape))


def _m16_of(mout):
    m4 = mout[0, 0:4] + mout[1, 0:4]
    m4 = jnp.where(m4 >= 0.0, m4, 0.2 * m4)
    return jnp.concatenate([m4, jnp.zeros((12,), jnp.float32)])


def kernel(x, edge_index, batch, u, params):
    s_arr = edge_index[0]
    d_arr = edge_index[1]
    n = x.shape[0]
    e = s_arr.shape[0]
    B = u.shape[0]
    nb = n // R
    f32 = jnp.float32
    p1, p2, p3 = params['g1'], params['g2'], params['g3']

    hw1, als1, ald1, mo1 = pl.pallas_call(
        _tc_pre_kernel,
        grid=(nb,),
        in_specs=[
            pl.BlockSpec((R, 16), lambda i: (i, 0)),
            _full((16, 64)), _full((1, 64)), _full((64, 128)),
            _full((128, 4)), _full((128, 4)),
        ],
        out_specs=[
            pl.BlockSpec((4, R, 32), lambda i: (0, i, 0)),
            pl.BlockSpec((R, 16), lambda i: (i, 0)),
            pl.BlockSpec((R, 16), lambda i: (i, 0)),
            _full((8, 128)),
        ],
        out_shape=[
            jax.ShapeDtypeStruct((4, n, 32), f32),
            jax.ShapeDtypeStruct((n, 16), f32),
            jax.ShapeDtypeStruct((n, 16), f32),
            jax.ShapeDtypeStruct((8, 128), f32),
        ],
    )(x, params['ne_W'], params['ne_b'].reshape(1, 64), p1['W'],
      _al_proj(p1['a_src']), _al_proj(p1['a_dst']))

    def sc_layer(hw, als16, ald16, mo, dph):
        m16 = _m16_of(mo)
        exr, den_p = _sc_pass_a(s_arr, d_arr, als16, ald16, m16, n, e)
        out_p = _sc_pass_b(s_arr, d_arr, exr, hw.reshape(4 * n, dph), n, e, dph)
        return m16, den_p, out_p

    m16_1, den1, out1 = sc_layer(hw1, als1, ald1, mo1, 32)

    def mid_call(dph_in, dph_out, out_p, den_p, hw, als16, ald16, m16,
                 pblk, wn, psn, pdn):
        return pl.pallas_call(
            functools.partial(_tc_mid_kernel, dph_in=dph_in, dph_out=dph_out),
            grid=(nb,),
            in_specs=[
                pl.BlockSpec((NC, 4, R, dph_in), lambda i: (0, 0, i, 0)),
                pl.BlockSpec((NC, R, 16), lambda i: (0, i, 0)),
                pl.BlockSpec((4, R, dph_in), lambda i: (0, i, 0)),
                pl.BlockSpec((R, 16), lambda i: (i, 0)),
                pl.BlockSpec((R, 16), lambda i: (i, 0)),
                _full((1, 16)),
                _full((1, 4 * dph_in)), _full((1, 4 * dph_in)), _full((1, 4 * dph_in)),
                _full((4 * dph_in, 4 * dph_out)),
                _full((4 * dph_out, 4)), _full((4 * dph_out, 4)),
            ],
            out_specs=[
                pl.BlockSpec((4, R, dph_out), lambda i: (0, i, 0)),
                pl.BlockSpec((R, 16), lambda i: (i, 0)),
                pl.BlockSpec((R, 16), lambda i: (i, 0)),
                _full((8, 128)),
            ],
            out_shape=[
                jax.ShapeDtypeStruct((4, n, dph_out), f32),
                jax.ShapeDtypeStruct((n, 16), f32),
                jax.ShapeDtypeStruct((n, 16), f32),
                jax.ShapeDtypeStruct((8, 128), f32),
            ],
        )(out_p, den_p, hw, als16, ald16, m16.reshape(1, 16),
          pblk['b'].reshape(1, -1), pblk['ln_g'].reshape(1, -1),
          pblk['ln_b'].reshape(1, -1), wn, psn, pdn)

    hw2, als2, ald2, mo2 = mid_call(32, 32, out1, den1, hw1, als1, ald1, m16_1,
                                    p1, p2['W'], _al_proj(p2['a_src']),
                                    _al_proj(p2['a_dst']))
    m16_2, den2, out2 = sc_layer(hw2, als2, ald2, mo2, 32)

    hw3, als3, ald3, mo3 = mid_call(32, 16, out2, den2, hw2, als2, ald2, m16_2,
                                    p2, p3['W'], _al_proj(p3['a_src']),
                                    _al_proj(p3['a_dst']))
    m16_3, den3, out3 = sc_layer(hw3, als3, ald3, mo3, 16)

    onehot = (batch[:, None] == jnp.arange(B, dtype=batch.dtype)[None, :]
              ).astype(f32)
    pooled = pl.pallas_call(
        _tc_post_kernel,
        grid=(nb,),
        in_specs=[
            pl.BlockSpec((NC, 4, R, 16), lambda i: (0, 0, i, 0)),
            pl.BlockSpec((NC, R, 16), lambda i: (0, i, 0)),
            pl.BlockSpec((4, R, 16), lambda i: (0, i, 0)),
            pl.BlockSpec((R, 16), lambda i: (i, 0)),
            pl.BlockSpec((R, 16), lambda i: (i, 0)),
            _full((1, 16)),
            _full((1, 64)), _full((1, 64)), _full((1, 64)),
            pl.BlockSpec((R, 32), lambda i: (i, 0)),
        ],
        out_specs=[_full((32, 128))],
        out_shape=[jax.ShapeDtypeStruct((32, 128), f32)],
    )(out3, den3, hw3, als3, ald3, m16_3.reshape(1, 16),
      p3['b'].reshape(1, -1), p3['ln_g'].reshape(1, -1),
      p3['ln_b'].reshape(1, -1), onehot)[0]

    pk = ['ge_W1', 'ge_b1', 'ge_g1', 'ge_be1', 'ge_W2', 'ge_b2', 'ge_g2', 'ge_be2',
          'fu_W1', 'fu_b1', 'fu_g1', 'fu_be1', 'fu_W2', 'fu_b2', 'fu_g2', 'fu_be2',
          'cl_W1', 'cl_b1', 'cl_W2', 'cl_b2']
    plist = [params[k] for k in pk]
    logits = pl.pallas_call(
        _head_kernel,
        out_shape=jax.ShapeDtypeStruct((B, 2), f32),
    )(pooled, u, *plist)
    return logits
